# Initial kernel scaffold; baseline (speedup 1.0000x reference)
import jax
import jax.numpy as jnp
from jax.experimental import pallas as pl


def kernel(x, W_qkv, W_proj, b_proj):
    raise NotImplementedError("write your pallas kernel here")


# speedup vs baseline: 1.3439x; 1.3439x over previous
import functools

import jax
import jax.numpy as jnp
from jax.experimental import pallas as pl

_H = 16
_NEG = -1e30


def _qkv_kernel(x_ref, w_ref, qkv_ref, sc_ref, *, H):
    xb = x_ref[...].astype(jnp.bfloat16)
    wb = w_ref[...].astype(jnp.bfloat16)
    acc = jax.lax.dot_general(
        xb, wb, (((1,), (0,)), ((), ())), preferred_element_type=jnp.float32
    )
    TM, C3 = acc.shape
    Ch = C3 // (3 * H)
    sq = acc[:, : C3 // 3] ** 2
    s = jnp.sqrt(sq.reshape(TM, H, Ch).sum(axis=2))
    sc_ref[...] = s.T[:, None, :]
    qkv_ref[...] = (
        acc.reshape(TM, 3 * H, Ch).transpose(1, 0, 2).astype(jnp.bfloat16)
    )


def _thresh_kernel(sc_ref, bias_ref, *, keep):
    s = sc_ref[...].reshape(sc_ref.shape[0], sc_ref.shape[2])
    si = jax.lax.bitcast_convert_type(s, jnp.int32)

    def body(_, lohi):
        lo, hi = lohi
        mid = lo + (hi - lo + 1) // 2
        cnt = jnp.sum((si >= mid).astype(jnp.int32), axis=1, keepdims=True)
        ok = cnt >= keep
        return jnp.where(ok, mid, lo), jnp.where(ok, hi, mid - 1)

    lo = jnp.zeros((si.shape[0], 1), jnp.int32)
    hi = jnp.full((si.shape[0], 1), 0x7F7FFFFF, jnp.int32)
    lo, _ = jax.lax.fori_loop(0, 31, body, (lo, hi))
    bias = jnp.where(si >= lo, 0.0, _NEG).astype(jnp.float32)
    bias_ref[...] = bias[:, None, :]


def _attn_kernel(q_ref, k_ref, v_ref, bias_ref, o_ref, *, scale):
    outs = []
    for hh in range(q_ref.shape[0]):
        q = q_ref[hh]
        k = k_ref[hh]
        logits = jax.lax.dot_general(
            q, k, (((1,), (1,)), ((), ())), preferred_element_type=jnp.float32
        ) * scale
        logits = jnp.clip(logits, -50.0, 50.0) + bias_ref[hh]
        m = jnp.max(logits, axis=1, keepdims=True)
        p = jnp.exp(logits - m)
        w = (p / jnp.sum(p, axis=1, keepdims=True)).astype(jnp.bfloat16)
        outs.append(
            jax.lax.dot_general(
                w, v_ref[hh], (((1,), (0,)), ((), ())),
                preferred_element_type=jnp.float32,
            ).astype(jnp.bfloat16)
        )
    o_ref[...] = jnp.concatenate(outs, axis=1)


def _proj_kernel(a_ref, w_ref, b_ref, o_ref):
    wb = w_ref[...].astype(jnp.bfloat16)
    acc = jax.lax.dot_general(
        a_ref[...], wb, (((1,), (0,)), ((), ())),
        preferred_element_type=jnp.float32,
    )
    o_ref[...] = jnp.clip(acc + b_ref[...], -10.0, 10.0)


def kernel(x, W_qkv, W_proj, b_proj):
    B, N, C = x.shape
    H = _H
    Ch = C // H
    BN = B * N
    keep = N // 2
    scale = Ch ** (-0.5)

    x2 = x.reshape(BN, C)
    Wq_t = W_qkv.T
    Wp_t = W_proj.T
    b2 = b_proj.reshape(1, C)

    TM = min(512, N)
    nrow = BN // TM
    ntile_b = N // TM

    qkv_t, scores = pl.pallas_call(
        functools.partial(_qkv_kernel, H=H),
        grid=(nrow,),
        in_specs=[
            pl.BlockSpec((TM, C), lambda g: (g, 0)),
            pl.BlockSpec((C, 3 * C), lambda g: (0, 0)),
        ],
        out_specs=[
            pl.BlockSpec((3 * H, TM, Ch), lambda g: (0, g, 0)),
            pl.BlockSpec((H, 1, TM), lambda g: (g // ntile_b, 0, g % ntile_b)),
        ],
        out_shape=[
            jax.ShapeDtypeStruct((3 * H, BN, Ch), jnp.bfloat16),
            jax.ShapeDtypeStruct((B * H, 1, N), jnp.float32),
        ],
    )(x2, Wq_t)

    bias = pl.pallas_call(
        functools.partial(_thresh_kernel, keep=keep),
        in_specs=[pl.BlockSpec((B * H, 1, N), lambda: (0, 0, 0))],
        out_specs=pl.BlockSpec((B * H, 1, N), lambda: (0, 0, 0)),
        out_shape=jax.ShapeDtypeStruct((B * H, 1, N), jnp.float32),
    )(scores)

    TMq = min(512, N)
    nq = N // TMq
    H2 = H // 2

    attn_out = pl.pallas_call(
        functools.partial(_attn_kernel, scale=scale),
        grid=(B, H2, nq),
        in_specs=[
            pl.BlockSpec((2, TMq, Ch), lambda b, h2, qt: (h2, b * nq + qt, 0)),
            pl.BlockSpec((2, N, Ch), lambda b, h2, qt: (H2 + h2, b, 0)),
            pl.BlockSpec((2, N, Ch), lambda b, h2, qt: (H + h2, b, 0)),
            pl.BlockSpec((2, 1, N), lambda b, h2, qt: (b * H2 + h2, 0, 0)),
        ],
        out_specs=pl.BlockSpec(
            (TMq, 2 * Ch), lambda b, h2, qt: (b * nq + qt, h2)
        ),
        out_shape=jax.ShapeDtypeStruct((BN, C), jnp.bfloat16),
    )(qkv_t, qkv_t, qkv_t, bias)

    out = pl.pallas_call(
        _proj_kernel,
        grid=(nrow,),
        in_specs=[
            pl.BlockSpec((TM, C), lambda g: (g, 0)),
            pl.BlockSpec((C, C), lambda g: (0, 0)),
            pl.BlockSpec((1, C), lambda g: (0, 0)),
        ],
        out_specs=pl.BlockSpec((TM, C), lambda g: (g, 0)),
        out_shape=jax.ShapeDtypeStruct((BN, C), jnp.float32),
    )(attn_out, Wp_t, b2)

    return out.reshape(B, N, C)


# speedup vs baseline: 1.5807x; 1.1762x over previous
import functools

import jax
import jax.numpy as jnp
from jax.experimental import pallas as pl

_H = 16
_NEG = -1e30


def _qkv_kernel(x_ref, w_ref, qkv_ref, sc_ref, *, H):
    xb = x_ref[...].astype(jnp.bfloat16)
    wb = w_ref[...].astype(jnp.bfloat16)
    acc = jax.lax.dot_general(
        xb, wb, (((1,), (1,)), ((), ())), preferred_element_type=jnp.float32
    )
    TM, C3 = acc.shape
    Ch = C3 // (3 * H)
    sq = acc[:, : C3 // 3] ** 2
    s = jnp.sqrt(sq.reshape(TM, H, Ch).sum(axis=2))
    sc_ref[...] = s.T[:, None, :]
    qkv_ref[...] = acc.astype(jnp.bfloat16)


def _thresh_kernel(sc_ref, bias_ref, *, keep):
    s = sc_ref[...].reshape(sc_ref.shape[0], sc_ref.shape[2])
    si = jax.lax.bitcast_convert_type(s, jnp.int32)

    def body(_, lohi):
        lo, hi = lohi
        mid = lo + (hi - lo + 1) // 2
        cnt = jnp.sum((si >= mid).astype(jnp.int32), axis=1, keepdims=True)
        ok = cnt >= keep
        return jnp.where(ok, mid, lo), jnp.where(ok, hi, mid - 1)

    lo = jnp.zeros((si.shape[0], 1), jnp.int32)
    hi = jnp.full((si.shape[0], 1), 0x7F7FFFFF, jnp.int32)
    lo, _ = jax.lax.fori_loop(0, 31, body, (lo, hi))
    bias = jnp.where(si >= lo, 0.0, _NEG).astype(jnp.float32)
    bias_ref[...] = bias[:, None, :]


def _attn_kernel(q_ref, k_ref, v_ref, bias_ref, o_ref, *, scale, Ch):
    outs = []
    for hh in range(2):
        sl = slice(hh * Ch, (hh + 1) * Ch)
        q = q_ref[:, sl]
        k = k_ref[:, sl]
        logits = jax.lax.dot_general(
            q, k, (((1,), (1,)), ((), ())), preferred_element_type=jnp.float32
        ) * scale
        logits = jnp.clip(logits, -50.0, 50.0) + bias_ref[hh]
        m = jnp.max(logits, axis=1, keepdims=True)
        p = jnp.exp(logits - m)
        w = (p / jnp.sum(p, axis=1, keepdims=True)).astype(jnp.bfloat16)
        outs.append(
            jax.lax.dot_general(
                w, v_ref[:, sl], (((1,), (0,)), ((), ())),
                preferred_element_type=jnp.float32,
            ).astype(jnp.bfloat16)
        )
    o_ref[...] = jnp.concatenate(outs, axis=1)


def _proj_kernel(a_ref, w_ref, b_ref, o_ref):
    wb = w_ref[...].astype(jnp.bfloat16)
    acc = jax.lax.dot_general(
        a_ref[...], wb, (((1,), (1,)), ((), ())),
        preferred_element_type=jnp.float32,
    )
    o_ref[...] = jnp.clip(acc + b_ref[...], -10.0, 10.0)


def kernel(x, W_qkv, W_proj, b_proj):
    B, N, C = x.shape
    H = _H
    Ch = C // H
    BN = B * N
    keep = N // 2
    scale = Ch ** (-0.5)

    x2 = x.reshape(BN, C)
    b2 = b_proj.reshape(1, C)

    TM = min(512, N)
    nrow = BN // TM
    ntile_b = N // TM

    qkv_flat, scores = pl.pallas_call(
        functools.partial(_qkv_kernel, H=H),
        grid=(nrow,),
        in_specs=[
            pl.BlockSpec((TM, C), lambda g: (g, 0)),
            pl.BlockSpec((3 * C, C), lambda g: (0, 0)),
        ],
        out_specs=[
            pl.BlockSpec((TM, 3 * C), lambda g: (g, 0)),
            pl.BlockSpec((H, 1, TM), lambda g: (g // ntile_b, 0, g % ntile_b)),
        ],
        out_shape=[
            jax.ShapeDtypeStruct((BN, 3 * C), jnp.bfloat16),
            jax.ShapeDtypeStruct((B * H, 1, N), jnp.float32),
        ],
    )(x2, W_qkv)

    bias = pl.pallas_call(
        functools.partial(_thresh_kernel, keep=keep),
        in_specs=[pl.BlockSpec((B * H, 1, N), lambda: (0, 0, 0))],
        out_specs=pl.BlockSpec((B * H, 1, N), lambda: (0, 0, 0)),
        out_shape=jax.ShapeDtypeStruct((B * H, 1, N), jnp.float32),
    )(scores)

    TMq = min(512, N)
    nq = N // TMq
    H2 = H // 2

    attn_out = pl.pallas_call(
        functools.partial(_attn_kernel, scale=scale, Ch=Ch),
        grid=(B, H2, nq),
        in_specs=[
            pl.BlockSpec((TMq, 2 * Ch), lambda b, h2, qt: (b * nq + qt, h2)),
            pl.BlockSpec((N, 2 * Ch), lambda b, h2, qt: (b, H2 + h2)),
            pl.BlockSpec((N, 2 * Ch), lambda b, h2, qt: (b, H + h2)),
            pl.BlockSpec((2, 1, N), lambda b, h2, qt: (b * H2 + h2, 0, 0)),
        ],
        out_specs=pl.BlockSpec(
            (TMq, 2 * Ch), lambda b, h2, qt: (b * nq + qt, h2)
        ),
        out_shape=jax.ShapeDtypeStruct((BN, C), jnp.bfloat16),
    )(qkv_flat, qkv_flat, qkv_flat, bias)

    out = pl.pallas_call(
        _proj_kernel,
        grid=(nrow,),
        in_specs=[
            pl.BlockSpec((TM, C), lambda g: (g, 0)),
            pl.BlockSpec((C, C), lambda g: (0, 0)),
            pl.BlockSpec((1, C), lambda g: (0, 0)),
        ],
        out_specs=pl.BlockSpec((TM, C), lambda g: (g, 0)),
        out_shape=jax.ShapeDtypeStruct((BN, C), jnp.float32),
    )(attn_out, W_proj, b2)

    return out.reshape(B, N, C)


# speedup vs baseline: 1.9917x; 1.2600x over previous
import functools

import jax
import jax.numpy as jnp
from jax.experimental import pallas as pl

_H = 16
_NEG = -1e30


def _qkv_kernel(x_ref, w_ref, qkv_ref, sc_ref, *, H):
    xb = x_ref[...].astype(jnp.bfloat16)
    acc = jax.lax.dot_general(
        xb, w_ref[...], (((1,), (1,)), ((), ())),
        preferred_element_type=jnp.float32,
    )
    TM, C3 = acc.shape
    Ch = C3 // (3 * H)
    sq = acc[:, : C3 // 3] ** 2
    s = jnp.sqrt(sq.reshape(TM, H, Ch).sum(axis=2))
    sc_ref[...] = s.T[:, None, :]
    qkv_ref[...] = acc.astype(jnp.bfloat16)


def _thresh_kernel(sc_ref, bias_ref, *, keep):
    s = sc_ref[...].reshape(sc_ref.shape[0], sc_ref.shape[2])
    si = jax.lax.bitcast_convert_type(s, jnp.int32)

    def body(_, lohi):
        lo, hi = lohi
        mid = lo + (hi - lo + 1) // 2
        cnt = jnp.sum((si >= mid).astype(jnp.int32), axis=1, keepdims=True)
        ok = cnt >= keep
        return jnp.where(ok, mid, lo), jnp.where(ok, hi, mid - 1)

    lo = jnp.zeros((si.shape[0], 1), jnp.int32)
    hi = jnp.full((si.shape[0], 1), 0x7F7FFFFF, jnp.int32)
    lo, _ = jax.lax.fori_loop(0, 31, body, (lo, hi))
    bias = jnp.where(si >= lo, 0.0, _NEG).astype(jnp.float32)
    bias_ref[...] = bias[:, None, :]


def _attn_kernel(q_ref, k_ref, v_ref, bias_ref, o_ref, *, scale, Ch):
    outs = []
    for hh in range(2):
        sl = slice(hh * Ch, (hh + 1) * Ch)
        q = q_ref[:, sl]
        k = k_ref[:, sl]
        logits = jax.lax.dot_general(
            q, k, (((1,), (1,)), ((), ())), preferred_element_type=jnp.float32
        ) * scale
        logits = jnp.clip(logits, -50.0, 50.0) + bias_ref[hh]
        p = jnp.exp(logits)
        w = (p * (1.0 / jnp.sum(p, axis=1, keepdims=True))).astype(jnp.bfloat16)
        outs.append(
            jax.lax.dot_general(
                w, v_ref[:, sl], (((1,), (0,)), ((), ())),
                preferred_element_type=jnp.float32,
            ).astype(jnp.bfloat16)
        )
    o_ref[...] = jnp.concatenate(outs, axis=1)


def _proj_kernel(a_ref, w_ref, b_ref, o_ref):
    acc = jax.lax.dot_general(
        a_ref[...], w_ref[...], (((1,), (1,)), ((), ())),
        preferred_element_type=jnp.float32,
    )
    o_ref[...] = jnp.clip(acc + b_ref[...], -10.0, 10.0)


def kernel(x, W_qkv, W_proj, b_proj):
    B, N, C = x.shape
    H = _H
    Ch = C // H
    BN = B * N
    keep = N // 2
    scale = Ch ** (-0.5)

    x2 = x.reshape(BN, C)
    wq_bf = W_qkv.astype(jnp.bfloat16)
    wp_bf = W_proj.astype(jnp.bfloat16)
    b2 = b_proj.reshape(1, C)

    TM = min(512, N)
    nrow = BN // TM
    ntile_b = N // TM

    qkv_flat, scores = pl.pallas_call(
        functools.partial(_qkv_kernel, H=H),
        grid=(nrow,),
        in_specs=[
            pl.BlockSpec((TM, C), lambda g: (g, 0)),
            pl.BlockSpec((3 * C, C), lambda g: (0, 0)),
        ],
        out_specs=[
            pl.BlockSpec((TM, 3 * C), lambda g: (g, 0)),
            pl.BlockSpec((H, 1, TM), lambda g: (g // ntile_b, 0, g % ntile_b)),
        ],
        out_shape=[
            jax.ShapeDtypeStruct((BN, 3 * C), jnp.bfloat16),
            jax.ShapeDtypeStruct((B * H, 1, N), jnp.float32),
        ],
    )(x2, wq_bf)

    bias = pl.pallas_call(
        functools.partial(_thresh_kernel, keep=keep),
        in_specs=[pl.BlockSpec((B * H, 1, N), lambda: (0, 0, 0))],
        out_specs=pl.BlockSpec((B * H, 1, N), lambda: (0, 0, 0)),
        out_shape=jax.ShapeDtypeStruct((B * H, 1, N), jnp.float32),
    )(scores)

    TMq = min(1024, N)
    nq = N // TMq
    H2 = H // 2

    attn_out = pl.pallas_call(
        functools.partial(_attn_kernel, scale=scale, Ch=Ch),
        grid=(B, H2, nq),
        in_specs=[
            pl.BlockSpec((TMq, 2 * Ch), lambda b, h2, qt: (b * nq + qt, h2)),
            pl.BlockSpec((N, 2 * Ch), lambda b, h2, qt: (b, H2 + h2)),
            pl.BlockSpec((N, 2 * Ch), lambda b, h2, qt: (b, H + h2)),
            pl.BlockSpec((2, 1, N), lambda b, h2, qt: (b * H2 + h2, 0, 0)),
        ],
        out_specs=pl.BlockSpec(
            (TMq, 2 * Ch), lambda b, h2, qt: (b * nq + qt, h2)
        ),
        out_shape=jax.ShapeDtypeStruct((BN, C), jnp.bfloat16),
    )(qkv_flat, qkv_flat, qkv_flat, bias)

    out = pl.pallas_call(
        _proj_kernel,
        grid=(nrow,),
        in_specs=[
            pl.BlockSpec((TM, C), lambda g: (g, 0)),
            pl.BlockSpec((C, C), lambda g: (0, 0)),
            pl.BlockSpec((1, C), lambda g: (0, 0)),
        ],
        out_specs=pl.BlockSpec((TM, C), lambda g: (g, 0)),
        out_shape=jax.ShapeDtypeStruct((BN, C), jnp.float32),
    )(attn_out, wp_bf, b2)

    return out.reshape(B, N, C)
